# Initial kernel scaffold; baseline (speedup 1.0000x reference)
import jax
import jax.numpy as jnp
from jax.experimental import pallas as pl


def kernel(deprel_path_edge, deparc_path_edge, path_len_edge, deprel_ext_path_edge, deprel_table, deparc_table):
    raise NotImplementedError("write your pallas kernel here")


# speedup vs baseline: 23.4923x; 23.4923x over previous
import functools

import jax
import jax.numpy as jnp
from jax import lax
from jax.experimental import pallas as pl
from jax.experimental.pallas import tpu as pltpu
from jax.experimental.pallas import tpu_sc as plsc

N = 320000
L = 8
DEPREL_SIZE = 50
DEPARC_SIZE = 4
DIM = 16

NC = 2
NS = 16
NW = NC * NS
RW = N // NW
C = 400
NCH = RW // C
NG = C // 16


def _sc_body(rel_hbm, arc_hbm, pair_hbm, quad_hbm, out_hbm,
             pair_v, quad_v, reli_v, arci_v, out_v):
  wid = lax.axis_index("s") * NC + lax.axis_index("c")
  pltpu.sync_copy(pair_hbm, pair_v)
  pltpu.sync_copy(quad_hbm, quad_v)
  row0 = wid * RW
  iota = lax.iota(jnp.int32, 16)

  def chunk_body(c, carry):
    r0 = row0 + c * C
    pltpu.sync_copy(rel_hbm.at[pl.ds(r0 * L, C * L)], reli_v)
    pltpu.sync_copy(arc_hbm.at[pl.ds(r0 * L, C * L)], arci_v)

    def group_body(g, carry2):
      base8 = (g * 16 + iota) * L
      p16 = []
      for k in range(4):
        e = plsc.load_gather(reli_v, [base8 + (2 * k)])
        o = plsc.load_gather(reli_v, [base8 + (2 * k + 1)])
        p16.append((e * DEPREL_SIZE + o) * DIM)
      a = [plsc.load_gather(arci_v, [base8 + l]) for l in range(L)]
      q0 = (((a[0] * 4 + a[1]) * 4 + a[2]) * 4 + a[3]) * DIM
      q1 = (((a[4] * 4 + a[5]) * 4 + a[6]) * 4 + a[7]) * DIM
      ob = (g * 16 + iota) * 32
      for d in range(DIM):
        v = (plsc.load_gather(pair_v, [p16[0] + d])
             + plsc.load_gather(pair_v, [p16[1] + d])
             + plsc.load_gather(pair_v, [p16[2] + d])
             + plsc.load_gather(pair_v, [p16[3] + d]))
        plsc.store_scatter(out_v, [ob + d], v)
        w = (plsc.load_gather(quad_v, [q0 + d])
             + plsc.load_gather(quad_v, [q1 + d]))
        plsc.store_scatter(out_v, [ob + 16 + d], w)
      return carry2

    lax.fori_loop(0, NG, group_body, 0)
    pltpu.sync_copy(out_v, out_hbm.at[pl.ds(r0 * 32, C * 32)])
    return carry

  lax.fori_loop(0, NCH, chunk_body, 0)


@jax.jit
def _encode(rel_idx, arc_idx, pair_rel, quad_arc):
  mesh = plsc.VectorSubcoreMesh(core_axis_name="c", subcore_axis_name="s")
  fn = pl.kernel(
      _sc_body,
      out_type=jax.ShapeDtypeStruct((N * 32,), jnp.float32),
      mesh=mesh,
      scratch_types=[
          pltpu.VMEM((DEPREL_SIZE * DEPREL_SIZE * DIM,), jnp.float32),
          pltpu.VMEM((256 * DIM,), jnp.float32),
          pltpu.VMEM((C * L,), jnp.int32),
          pltpu.VMEM((C * L,), jnp.int32),
          pltpu.VMEM((C * 32,), jnp.float32),
      ],
      compiler_params=pltpu.CompilerParams(needs_layout_passes=False),
  )
  return fn(rel_idx, arc_idx, pair_rel, quad_arc)


def kernel(deprel_path_edge, deparc_path_edge, path_len_edge,
           deprel_ext_path_edge, deprel_table, deparc_table):
  del path_len_edge, deprel_ext_path_edge
  rel_idx = deprel_path_edge.astype(jnp.int32).reshape(-1)
  arc_idx = deparc_path_edge.astype(jnp.int32).reshape(-1)
  pair_rel = (deprel_table[:, None, :] + deprel_table[None, :, :]).reshape(-1)
  s2 = (deparc_table[:, None, :] + deparc_table[None, :, :]).reshape(16, DIM)
  quad_arc = (s2[:, None, :] + s2[None, :, :]).reshape(-1)
  out = _encode(rel_idx, arc_idx, pair_rel, quad_arc)
  return out.reshape(N, 32)


# speedup vs baseline: 38.9950x; 1.6599x over previous
import functools

import jax
import jax.numpy as jnp
from jax import lax
from jax.experimental import pallas as pl
from jax.experimental.pallas import tpu as pltpu
from jax.experimental.pallas import tpu_sc as plsc

N = 320000
L = 8
DEPREL_SIZE = 50
DIM = 16

NC = 2
NS = 16
NW = NC * NS
RW = N // NW
C = 1000
NCH = RW // C


def _take(v, idx):
  return v.at[idx].get(mode="promise_in_bounds")


def _sc_body(rel_hbm, arc_hbm, pair_hbm, quad_hbm, out_hbm,
             pair_v, quad_v, reli_v, arci_v, out_v):
  wid = lax.axis_index("s") * NC + lax.axis_index("c")
  pltpu.sync_copy(pair_hbm, pair_v)
  pltpu.sync_copy(quad_hbm, quad_v)
  row0 = wid * RW
  iota = lax.iota(jnp.int32, 16)
  ev8 = (iota & 7) * 2
  od8 = ev8 + 1
  ev4 = (iota & 3) * 2
  od4 = ev4 + 1
  zero = iota & 0

  def bcast(v, k):
    return _take(v, zero + k)

  def chunk_body(c, carry):
    r0 = row0 + c * C
    pltpu.sync_copy(rel_hbm.at[pl.ds(r0 * L, C * L)], reli_v)
    pltpu.sync_copy(arc_hbm.at[pl.ds(r0 * L, C * L)], arci_v)

    def pair_body(j, carry2):
      vb = j * 16
      vrel = reli_v[pl.ds(vb, 16)]
      varc = arci_v[pl.ds(vb, 16)]
      e = _take(vrel, ev8)
      o = _take(vrel, od8)
      p16 = (e * DEPREL_SIZE + o) * DIM
      e1 = _take(varc, ev8)
      o1 = _take(varc, od8)
      pp = e1 * 4 + o1
      e2 = _take(pp, ev4)
      o2 = _take(pp, od4)
      q16 = (e2 * 16 + o2) * DIM

      def row_sum(base_lanes, qlanes):
        acc = plsc.load_gather(pair_v, [bcast(p16, base_lanes[0]) + iota])
        for k in base_lanes[1:]:
          acc = acc + plsc.load_gather(pair_v, [bcast(p16, k) + iota])
        accq = plsc.load_gather(quad_v, [bcast(q16, qlanes[0]) + iota])
        accq = accq + plsc.load_gather(quad_v, [bcast(q16, qlanes[1]) + iota])
        return acc, accq

      ob = j * 64
      ra, qa = row_sum([0, 1, 2, 3], [0, 1])
      out_v[pl.ds(ob, 16)] = ra
      out_v[pl.ds(ob + 16, 16)] = qa
      rb, qb = row_sum([4, 5, 6, 7], [2, 3])
      out_v[pl.ds(ob + 32, 16)] = rb
      out_v[pl.ds(ob + 48, 16)] = qb
      return carry2

    lax.fori_loop(0, C // 2, pair_body, 0)
    pltpu.sync_copy(out_v, out_hbm.at[pl.ds(r0 * 32, C * 32)])
    return carry

  lax.fori_loop(0, NCH, chunk_body, 0)


@jax.jit
def _encode(rel_idx, arc_idx, pair_rel, quad_arc):
  mesh = plsc.VectorSubcoreMesh(core_axis_name="c", subcore_axis_name="s")
  fn = pl.kernel(
      _sc_body,
      out_type=jax.ShapeDtypeStruct((N * 32,), jnp.float32),
      mesh=mesh,
      scratch_types=[
          pltpu.VMEM((DEPREL_SIZE * DEPREL_SIZE * DIM,), jnp.float32),
          pltpu.VMEM((256 * DIM,), jnp.float32),
          pltpu.VMEM((C * L,), jnp.int32),
          pltpu.VMEM((C * L,), jnp.int32),
          pltpu.VMEM((C * 32,), jnp.float32),
      ],
      compiler_params=pltpu.CompilerParams(needs_layout_passes=False),
  )
  return fn(rel_idx, arc_idx, pair_rel, quad_arc)


def kernel(deprel_path_edge, deparc_path_edge, path_len_edge,
           deprel_ext_path_edge, deprel_table, deparc_table):
  del path_len_edge, deprel_ext_path_edge
  rel_idx = deprel_path_edge.astype(jnp.int32).reshape(-1)
  arc_idx = deparc_path_edge.astype(jnp.int32).reshape(-1)
  pair_rel = (deprel_table[:, None, :] + deprel_table[None, :, :]).reshape(-1)
  s2 = (deparc_table[:, None, :] + deparc_table[None, :, :]).reshape(16, DIM)
  quad_arc = (s2[:, None, :] + s2[None, :, :]).reshape(-1)
  out = _encode(rel_idx, arc_idx, pair_rel, quad_arc)
  return out.reshape(N, 32)


# speedup vs baseline: 46.1915x; 1.1846x over previous
import functools

import jax
import jax.numpy as jnp
from jax import lax
from jax.experimental import pallas as pl
from jax.experimental.pallas import tpu as pltpu
from jax.experimental.pallas import tpu_sc as plsc

N = 320000
L = 8
DEPREL_SIZE = 50
DIM = 16

NC = 2
NS = 16
NW = NC * NS
RW = N // NW
C = 1000
NCH = RW // C


def _take(v, idx):
  return v.at[idx].get(mode="promise_in_bounds")


def _sc_body(rel_hbm, arc_hbm, pair_hbm, quad_hbm, out_hbm,
             pair_v, quad_v, reli_v, arci_v, out_v):
  wid = lax.axis_index("s") * NC + lax.axis_index("c")
  pltpu.sync_copy(pair_hbm, pair_v)
  pltpu.sync_copy(quad_hbm, quad_v)
  row0 = wid * RW
  iota = lax.iota(jnp.int32, 16)
  ev8 = (iota & 7) * 2
  od8 = ev8 + 1
  ev4 = (iota & 3) * 2
  od4 = ev4 + 1
  zero = iota & 0

  def bcast(v, k):
    return _take(v, zero + k)

  def chunk_body(c, carry):
    r0 = row0 + c * C
    pltpu.sync_copy(rel_hbm.at[pl.ds(r0 * L, C * L)], reli_v)
    pltpu.sync_copy(arc_hbm.at[pl.ds(r0 * L, C * L)], arci_v)

    def pair_body(j):
      vb = j * 16
      vrel = reli_v[pl.ds(vb, 16)]
      varc = arci_v[pl.ds(vb, 16)]
      e = _take(vrel, ev8)
      o = _take(vrel, od8)
      p16 = (e * DEPREL_SIZE + o) * DIM
      e1 = _take(varc, ev8)
      o1 = _take(varc, od8)
      pp = e1 * 4 + o1
      e2 = _take(pp, ev4)
      o2 = _take(pp, od4)
      q16 = (e2 * 16 + o2) * DIM

      def row_sum(base_lanes, qlanes):
        acc = plsc.load_gather(pair_v, [bcast(p16, base_lanes[0]) + iota])
        for k in base_lanes[1:]:
          acc = acc + plsc.load_gather(pair_v, [bcast(p16, k) + iota])
        accq = plsc.load_gather(quad_v, [bcast(q16, qlanes[0]) + iota])
        accq = accq + plsc.load_gather(quad_v, [bcast(q16, qlanes[1]) + iota])
        return acc, accq

      ob = j * 64
      ra, qa = row_sum([0, 1, 2, 3], [0, 1])
      out_v[pl.ds(ob, 16)] = ra
      out_v[pl.ds(ob + 16, 16)] = qa
      rb, qb = row_sum([4, 5, 6, 7], [2, 3])
      out_v[pl.ds(ob + 32, 16)] = rb
      out_v[pl.ds(ob + 48, 16)] = qb

    plsc.parallel_loop(0, C // 2, unroll=4)(pair_body)
    pltpu.sync_copy(out_v, out_hbm.at[pl.ds(r0 * 32, C * 32)])
    return carry

  lax.fori_loop(0, NCH, chunk_body, 0)


@jax.jit
def _encode(rel_idx, arc_idx, pair_rel, quad_arc):
  mesh = plsc.VectorSubcoreMesh(core_axis_name="c", subcore_axis_name="s")
  fn = pl.kernel(
      _sc_body,
      out_type=jax.ShapeDtypeStruct((N * 32,), jnp.float32),
      mesh=mesh,
      scratch_types=[
          pltpu.VMEM((DEPREL_SIZE * DEPREL_SIZE * DIM,), jnp.float32),
          pltpu.VMEM((256 * DIM,), jnp.float32),
          pltpu.VMEM((C * L,), jnp.int32),
          pltpu.VMEM((C * L,), jnp.int32),
          pltpu.VMEM((C * 32,), jnp.float32),
      ],
      compiler_params=pltpu.CompilerParams(needs_layout_passes=False),
  )
  return fn(rel_idx, arc_idx, pair_rel, quad_arc)


def kernel(deprel_path_edge, deparc_path_edge, path_len_edge,
           deprel_ext_path_edge, deprel_table, deparc_table):
  del path_len_edge, deprel_ext_path_edge
  rel_idx = deprel_path_edge.astype(jnp.int32).reshape(-1)
  arc_idx = deparc_path_edge.astype(jnp.int32).reshape(-1)
  pair_rel = (deprel_table[:, None, :] + deprel_table[None, :, :]).reshape(-1)
  s2 = (deparc_table[:, None, :] + deparc_table[None, :, :]).reshape(16, DIM)
  quad_arc = (s2[:, None, :] + s2[None, :, :]).reshape(-1)
  out = _encode(rel_idx, arc_idx, pair_rel, quad_arc)
  return out.reshape(N, 32)


# speedup vs baseline: 46.2754x; 1.0018x over previous
import functools

import jax
import jax.numpy as jnp
from jax import lax
from jax.experimental import pallas as pl
from jax.experimental.pallas import tpu as pltpu
from jax.experimental.pallas import tpu_sc as plsc

N = 320000
L = 8
DEPREL_SIZE = 50
DIM = 16
PAIR_ROWS = 2560

NC = 2
NS = 16
NW = NC * NS
C = 1280
NCHT = N // C
IR = C * L // 128
OR = C * 32 // 128
MAXCH = -(-NCHT // NW)


def _take(v, idx):
  return v.at[idx].get(mode="promise_in_bounds")


def _sc_body(rel_hbm, arc_hbm, pair_hbm, quad_hbm, out_hbm,
             reli_v, arci_v, out_v, pair_v, quad_v):
  wid = lax.axis_index("s") * NC + lax.axis_index("c")
  pltpu.sync_copy(pair_hbm, pair_v)
  pltpu.sync_copy(quad_hbm, quad_v)
  iota = lax.iota(jnp.int32, 16)
  ev8 = (iota & 7) * 2
  od8 = ev8 + 1
  ev4 = (iota & 3) * 2
  od4 = ev4 + 1
  zero = iota & 0

  def bcast(v, k):
    return _take(v, zero + k)

  def chunk_body(c, carry):
    k = c * NW + wid

    @pl.when(k < NCHT)
    def _():
      pltpu.sync_copy(rel_hbm.at[pl.ds(k * IR, IR)], reli_v)
      pltpu.sync_copy(arc_hbm.at[pl.ds(k * IR, IR)], arci_v)

      def pair_body(j):
        ir = j >> 3
        ic = (j & 7) << 4
        vrel = reli_v[ir, pl.ds(ic, 16)]
        varc = arci_v[ir, pl.ds(ic, 16)]
        e = _take(vrel, ev8)
        o = _take(vrel, od8)
        p16 = (e * DEPREL_SIZE + o) * DIM
        e1 = _take(varc, ev8)
        o1 = _take(varc, od8)
        pp = e1 * 4 + o1
        e2 = _take(pp, ev4)
        o2 = _take(pp, od4)
        q16 = (e2 * 16 + o2) * DIM

        def row_sum(base_lanes, qlanes):
          acc = plsc.load_gather(pair_v, [bcast(p16, base_lanes[0]) + iota])
          for m in base_lanes[1:]:
            acc = acc + plsc.load_gather(pair_v, [bcast(p16, m) + iota])
          accq = plsc.load_gather(quad_v, [bcast(q16, qlanes[0]) + iota])
          accq = accq + plsc.load_gather(quad_v, [bcast(q16, qlanes[1]) + iota])
          return acc, accq

        orow = j >> 1
        oc = (j & 1) << 6
        ra, qa = row_sum([0, 1, 2, 3], [0, 1])
        out_v[orow, pl.ds(oc, 16)] = ra
        out_v[orow, pl.ds(oc + 16, 16)] = qa
        rb, qb = row_sum([4, 5, 6, 7], [2, 3])
        out_v[orow, pl.ds(oc + 32, 16)] = rb
        out_v[orow, pl.ds(oc + 48, 16)] = qb

      plsc.parallel_loop(0, C // 2, unroll=4)(pair_body)
      pltpu.sync_copy(out_v, out_hbm.at[pl.ds(k * OR, OR)])

    return carry

  lax.fori_loop(0, MAXCH, chunk_body, 0)


@jax.jit
def _encode(rel_idx, arc_idx, pair_rel, quad_arc):
  mesh = plsc.VectorSubcoreMesh(core_axis_name="c", subcore_axis_name="s")
  fn = pl.kernel(
      _sc_body,
      out_type=jax.ShapeDtypeStruct((N * 32 // 128, 128), jnp.float32),
      mesh=mesh,
      scratch_types=[
          pltpu.VMEM((IR, 128), jnp.int32),
          pltpu.VMEM((IR, 128), jnp.int32),
          pltpu.VMEM((OR, 128), jnp.float32),
          pltpu.VMEM((PAIR_ROWS * DIM,), jnp.float32),
          pltpu.VMEM((256 * DIM,), jnp.float32),
      ],
      compiler_params=pltpu.CompilerParams(needs_layout_passes=False,
                                           use_tc_tiling_on_sc=True),
  )
  return fn(rel_idx, arc_idx, pair_rel, quad_arc)


def kernel(deprel_path_edge, deparc_path_edge, path_len_edge,
           deprel_ext_path_edge, deprel_table, deparc_table):
  del path_len_edge, deprel_ext_path_edge
  rel_idx = deprel_path_edge.astype(jnp.int32).reshape(N * L // 128, 128)
  arc_idx = deparc_path_edge.astype(jnp.int32).reshape(N * L // 128, 128)
  pair_rel = (deprel_table[:, None, :] + deprel_table[None, :, :]).reshape(-1)
  pair_rel = jnp.pad(pair_rel, (0, (PAIR_ROWS - 2500) * DIM))
  s2 = (deparc_table[:, None, :] + deparc_table[None, :, :]).reshape(16, DIM)
  quad_arc = (s2[:, None, :] + s2[None, :, :]).reshape(-1)
  out = _encode(rel_idx, arc_idx, pair_rel, quad_arc)
  return out.reshape(N, 32)


# speedup vs baseline: 73.5547x; 1.5895x over previous
import functools

import jax
import jax.numpy as jnp
from jax import lax
from jax.experimental import pallas as pl
from jax.experimental.pallas import tpu as pltpu
from jax.experimental.pallas import tpu_sc as plsc

N = 320000
L = 8
DEPREL_SIZE = 50
DIM = 16
PAIR_ROWS = 2560

NC = 2
NS = 16
NW = NC * NS
NB = N // 128
BP = 4
CHW = BP * 1024
NCHT = NB // BP
MAXCH = -(-NCHT // NW)
TREG = NB * 1024


def _sc_body(rel_hbm, arc_hbm, pair_hbm, quad_hbm, out_hbm,
             reli_v, arci_v, out_v, pair_v, quad_v):
  wid = lax.axis_index("s") * NC + lax.axis_index("c")
  pltpu.sync_copy(pair_hbm, pair_v)
  pltpu.sync_copy(quad_hbm, quad_v)
  iota = lax.iota(jnp.int32, 16)

  def chunk_body(c, carry):
    k = c * NW + wid

    @pl.when(k < NCHT)
    def _():
      pltpu.sync_copy(rel_hbm.at[pl.ds(k * CHW, CHW)], reli_v)
      pltpu.sync_copy(arc_hbm.at[pl.ds(k * CHW, CHW)], arci_v)

      for b in range(BP):

        def grp(g, b=b):
          n16 = g * 16
          colv = iota + n16
          rl = [reli_v[pl.ds(b * 1024 + l * 128 + n16, 16)] for l in range(L)]
          al = [arci_v[pl.ds(b * 1024 + l * 128 + n16, 16)] for l in range(L)]
          p16 = [(rl[2 * m] * DEPREL_SIZE + rl[2 * m + 1]) * DIM
                 for m in range(4)]
          q16 = [((((al[4 * m] * 4 + al[4 * m + 1]) * 4
                    + al[4 * m + 2]) * 4 + al[4 * m + 3]) * DIM)
                 for m in range(2)]
          for d0 in range(DIM):
            dp = (iota + d0) & 15
            accr = (plsc.load_gather(pair_v, [p16[0] + dp])
                    + plsc.load_gather(pair_v, [p16[1] + dp])
                    + plsc.load_gather(pair_v, [p16[2] + dp])
                    + plsc.load_gather(pair_v, [p16[3] + dp]))
            acca = (plsc.load_gather(quad_v, [q16[0] + dp])
                    + plsc.load_gather(quad_v, [q16[1] + dp]))
            af = (((dp >> 3) << 12) + ((dp & 7) << 7)
                  + (b * 1024) + colv)
            plsc.store_scatter(out_v, [af], accr)
            plsc.store_scatter(out_v, [af + 2 * BP * 1024], acca)

        plsc.parallel_loop(0, 8, unroll=2)(grp)

      for t in range(4):
        pltpu.sync_copy(out_v.at[pl.ds(t * CHW, CHW)],
                        out_hbm.at[pl.ds(t * TREG + k * CHW, CHW)])

    return carry

  lax.fori_loop(0, MAXCH, chunk_body, 0)


@jax.jit
def _encode(rel_idx, arc_idx, pair_rel, quad_arc):
  mesh = plsc.VectorSubcoreMesh(core_axis_name="c", subcore_axis_name="s")
  fn = pl.kernel(
      _sc_body,
      out_type=jax.ShapeDtypeStruct((4 * TREG,), jnp.float32),
      mesh=mesh,
      scratch_types=[
          pltpu.VMEM((CHW,), jnp.int32),
          pltpu.VMEM((CHW,), jnp.int32),
          pltpu.VMEM((4 * CHW,), jnp.float32),
          pltpu.VMEM((PAIR_ROWS * DIM,), jnp.float32),
          pltpu.VMEM((256 * DIM,), jnp.float32),
      ],
      compiler_params=pltpu.CompilerParams(needs_layout_passes=False),
  )
  return fn(rel_idx, arc_idx, pair_rel, quad_arc)


def kernel(deprel_path_edge, deparc_path_edge, path_len_edge,
           deprel_ext_path_edge, deprel_table, deparc_table):
  del path_len_edge, deprel_ext_path_edge
  rel_idx = (deprel_path_edge.astype(jnp.int32)
             .reshape(NB, 128, L).transpose(0, 2, 1).reshape(-1))
  arc_idx = (deparc_path_edge.astype(jnp.int32)
             .reshape(NB, 128, L).transpose(0, 2, 1).reshape(-1))
  pair_rel = (deprel_table[:, None, :] + deprel_table[None, :, :]).reshape(-1)
  pair_rel = jnp.pad(pair_rel, (0, (PAIR_ROWS - 2500) * DIM))
  s2 = (deparc_table[:, None, :] + deparc_table[None, :, :]).reshape(16, DIM)
  quad_arc = (s2[:, None, :] + s2[None, :, :]).reshape(-1)
  out = _encode(rel_idx, arc_idx, pair_rel, quad_arc)
  return out.reshape(4, NB, 8, 128).transpose(1, 3, 0, 2).reshape(N, 32)


# speedup vs baseline: 173.9675x; 2.3651x over previous
import functools

import jax
import jax.numpy as jnp
from jax import lax
from jax.experimental import pallas as pl
from jax.experimental.pallas import tpu as pltpu
from jax.experimental.pallas import tpu_sc as plsc

N = 320000
L = 8
DEPREL_SIZE = 50
DIM = 16
PAIR_ROWS = 2560

NC = 2
NS = 16
NW = NC * NS
NB = N // 128
BP = 4
CHW = BP * 1024
NCHT = NB // BP
MAXCH = -(-NCHT // NW)
TREG = NB * 1024


def _sc_body(rel_hbm, arc_hbm, pair_hbm, quad_hbm, out_hbm,
             reli_v, arci_v, out_v, pair_v, quad_v, in_sem, out_sem):
  wid = lax.axis_index("s") * NC + lax.axis_index("c")
  pltpu.sync_copy(pair_hbm, pair_v)
  pltpu.sync_copy(quad_hbm, quad_v)
  iota = lax.iota(jnp.int32, 16)

  def issue_in(k, par):
    pltpu.async_copy(rel_hbm.at[pl.ds(k * CHW, CHW)],
                     reli_v.at[pl.ds(par * CHW, CHW)], in_sem)
    pltpu.async_copy(arc_hbm.at[pl.ds(k * CHW, CHW)],
                     arci_v.at[pl.ds(par * CHW, CHW)], in_sem)

  def drain_in():
    pltpu.make_async_copy(rel_hbm.at[pl.ds(0, CHW)],
                          reli_v.at[pl.ds(0, CHW)], in_sem).wait()
    pltpu.make_async_copy(arc_hbm.at[pl.ds(0, CHW)],
                          arci_v.at[pl.ds(0, CHW)], in_sem).wait()

  def drain_out():
    for t in range(4):
      pltpu.make_async_copy(out_hbm.at[pl.ds(0, CHW)],
                            out_v.at[pl.ds(t * CHW, CHW)], out_sem).wait()

  issue_in(wid, 0)

  def chunk_body(c, carry):
    k = c * NW + wid
    par = c & 1

    @pl.when(k < NCHT)
    def _():
      drain_in()

      @pl.when(k + NW < NCHT)
      def _():
        issue_in(k + NW, 1 - par)

      @pl.when(c >= 2)
      def _():
        drain_out()

      ibase = par * CHW
      obase = par * 4 * CHW

      for b in range(BP):

        def grp(g, b=b):
          n16 = g * 16
          colv = iota + (n16 + obase)
          rl = [reli_v[pl.ds(ibase + b * 1024 + l * 128 + n16, 16)]
                for l in range(L)]
          al = [arci_v[pl.ds(ibase + b * 1024 + l * 128 + n16, 16)]
                for l in range(L)]
          p16 = [(rl[2 * m] * DEPREL_SIZE + rl[2 * m + 1]) * DIM
                 for m in range(4)]
          q16 = [((((al[4 * m] * 4 + al[4 * m + 1]) * 4
                    + al[4 * m + 2]) * 4 + al[4 * m + 3]) * DIM)
                 for m in range(2)]
          for d0 in range(DIM):
            dp = (iota + d0) & 15
            accr = (plsc.load_gather(pair_v, [p16[0] + dp])
                    + plsc.load_gather(pair_v, [p16[1] + dp])
                    + plsc.load_gather(pair_v, [p16[2] + dp])
                    + plsc.load_gather(pair_v, [p16[3] + dp]))
            acca = (plsc.load_gather(quad_v, [q16[0] + dp])
                    + plsc.load_gather(quad_v, [q16[1] + dp]))
            af = (((dp >> 3) << 12) + ((dp & 7) << 7)
                  + (b * 1024) + colv)
            plsc.store_scatter(out_v, [af], accr)
            plsc.store_scatter(out_v, [af + 2 * BP * 1024], acca)

        plsc.parallel_loop(0, 8, unroll=1)(grp)

      for t in range(4):
        pltpu.async_copy(out_v.at[pl.ds(obase + t * CHW, CHW)],
                         out_hbm.at[pl.ds(t * TREG + k * CHW, CHW)], out_sem)

    return carry

  lax.fori_loop(0, MAXCH, chunk_body, 0)
  drain_out()
  drain_out()


@jax.jit
def _encode(rel_idx, arc_idx, pair_rel, quad_arc):
  mesh = plsc.VectorSubcoreMesh(core_axis_name="c", subcore_axis_name="s")
  fn = pl.kernel(
      _sc_body,
      out_type=jax.ShapeDtypeStruct((4 * TREG,), jnp.float32),
      mesh=mesh,
      scratch_types=[
          pltpu.VMEM((2 * CHW,), jnp.int32),
          pltpu.VMEM((2 * CHW,), jnp.int32),
          pltpu.VMEM((2 * 4 * CHW,), jnp.float32),
          pltpu.VMEM((PAIR_ROWS * DIM,), jnp.float32),
          pltpu.VMEM((256 * DIM,), jnp.float32),
          pltpu.SemaphoreType.DMA,
          pltpu.SemaphoreType.DMA,
      ],
      compiler_params=pltpu.CompilerParams(needs_layout_passes=False),
  )
  return fn(rel_idx, arc_idx, pair_rel, quad_arc)


def kernel(deprel_path_edge, deparc_path_edge, path_len_edge,
           deprel_ext_path_edge, deprel_table, deparc_table):
  del path_len_edge, deprel_ext_path_edge
  rel_idx = (deprel_path_edge.astype(jnp.int32)
             .reshape(NB, 128, L).transpose(0, 2, 1).reshape(-1))
  arc_idx = (deparc_path_edge.astype(jnp.int32)
             .reshape(NB, 128, L).transpose(0, 2, 1).reshape(-1))
  pair_rel = (deprel_table[:, None, :] + deprel_table[None, :, :]).reshape(-1)
  pair_rel = jnp.pad(pair_rel, (0, (PAIR_ROWS - 2500) * DIM))
  s2 = (deparc_table[:, None, :] + deparc_table[None, :, :]).reshape(16, DIM)
  quad_arc = (s2[:, None, :] + s2[None, :, :]).reshape(-1)
  out = _encode(rel_idx, arc_idx, pair_rel, quad_arc)
  return out.reshape(4, NB, 8, 128).transpose(1, 3, 0, 2).reshape(N, 32)


# speedup vs baseline: 175.3022x; 1.0077x over previous
import functools

import jax
import jax.numpy as jnp
from jax import lax
from jax.experimental import pallas as pl
from jax.experimental.pallas import tpu as pltpu
from jax.experimental.pallas import tpu_sc as plsc

N = 320000
L = 8
DEPREL_SIZE = 50
DIM = 16
PAIR_ROWS = 2560

NC = 2
NS = 16
NW = NC * NS
NB = N // 128
BP = 4
CHW = BP * 1024
NCHT = NB // BP
MAXCH = -(-NCHT // NW)
TREG = NB * 1024


def _sc_body(rel_hbm, arc_hbm, pair_hbm, quad_hbm, out_hbm,
             reli_v, arci_v, out_v, pair_v, quad_v, in_sem, out_sem):
  wid = lax.axis_index("s") * NC + lax.axis_index("c")
  iota = lax.iota(jnp.int32, 16)

  def issue_in(k, par):
    pltpu.async_copy(rel_hbm.at[pl.ds(k * CHW, CHW)],
                     reli_v.at[pl.ds(par * CHW, CHW)], in_sem)
    pltpu.async_copy(arc_hbm.at[pl.ds(k * CHW, CHW)],
                     arci_v.at[pl.ds(par * CHW, CHW)], in_sem)

  def drain_in():
    pltpu.make_async_copy(rel_hbm.at[pl.ds(0, CHW)],
                          reli_v.at[pl.ds(0, CHW)], in_sem).wait()
    pltpu.make_async_copy(arc_hbm.at[pl.ds(0, CHW)],
                          arci_v.at[pl.ds(0, CHW)], in_sem).wait()

  def drain_out():
    for t in range(4):
      pltpu.make_async_copy(out_hbm.at[pl.ds(0, CHW)],
                            out_v.at[pl.ds(t * CHW, CHW)], out_sem).wait()

  issue_in(wid, 0)
  pltpu.sync_copy(pair_hbm, pair_v)
  pltpu.sync_copy(quad_hbm, quad_v)

  def chunk_body(c, carry):
    k = c * NW + wid
    par = c & 1

    @pl.when(k < NCHT)
    def _():
      drain_in()

      @pl.when(k + NW < NCHT)
      def _():
        issue_in(k + NW, 1 - par)

      @pl.when(c >= 2)
      def _():
        drain_out()

      ibase = par * CHW
      obase = par * 4 * CHW

      for b in range(BP):

        def grp(g, b=b):
          n16 = g * 16
          colv = iota + (n16 + obase)
          rl = [reli_v[pl.ds(ibase + b * 1024 + l * 128 + n16, 16)]
                for l in range(L)]
          al = [arci_v[pl.ds(ibase + b * 1024 + l * 128 + n16, 16)]
                for l in range(L)]
          p16 = [(rl[2 * m] * DEPREL_SIZE + rl[2 * m + 1]) * DIM
                 for m in range(4)]
          q16 = [((((al[4 * m] * 4 + al[4 * m + 1]) * 4
                    + al[4 * m + 2]) * 4 + al[4 * m + 3]) * DIM)
                 for m in range(2)]
          for d0 in range(DIM):
            dp = (iota + d0) & 15
            accr = (plsc.load_gather(pair_v, [p16[0] + dp])
                    + plsc.load_gather(pair_v, [p16[1] + dp])
                    + plsc.load_gather(pair_v, [p16[2] + dp])
                    + plsc.load_gather(pair_v, [p16[3] + dp]))
            acca = (plsc.load_gather(quad_v, [q16[0] + dp])
                    + plsc.load_gather(quad_v, [q16[1] + dp]))
            af = (((dp >> 3) << 12) + ((dp & 7) << 7)
                  + (b * 1024) + colv)
            plsc.store_scatter(out_v, [af], accr)
            plsc.store_scatter(out_v, [af + 2 * BP * 1024], acca)

        plsc.parallel_loop(0, 8, unroll=1)(grp)

      for t in range(4):
        pltpu.async_copy(out_v.at[pl.ds(obase + t * CHW, CHW)],
                         out_hbm.at[pl.ds(t * TREG + k * CHW, CHW)], out_sem)

    return carry

  lax.fori_loop(0, MAXCH, chunk_body, 0)
  drain_out()
  drain_out()


@jax.jit
def _encode(rel_idx, arc_idx, pair_rel, quad_arc):
  mesh = plsc.VectorSubcoreMesh(core_axis_name="c", subcore_axis_name="s")
  fn = pl.kernel(
      _sc_body,
      out_type=jax.ShapeDtypeStruct((4 * TREG,), jnp.float32),
      mesh=mesh,
      scratch_types=[
          pltpu.VMEM((2 * CHW,), jnp.int32),
          pltpu.VMEM((2 * CHW,), jnp.int32),
          pltpu.VMEM((2 * 4 * CHW,), jnp.float32),
          pltpu.VMEM((PAIR_ROWS * DIM,), jnp.float32),
          pltpu.VMEM((256 * DIM,), jnp.float32),
          pltpu.SemaphoreType.DMA,
          pltpu.SemaphoreType.DMA,
      ],
      compiler_params=pltpu.CompilerParams(needs_layout_passes=False),
  )
  return fn(rel_idx, arc_idx, pair_rel, quad_arc)


def kernel(deprel_path_edge, deparc_path_edge, path_len_edge,
           deprel_ext_path_edge, deprel_table, deparc_table):
  del path_len_edge, deprel_ext_path_edge
  rel_idx = (deprel_path_edge.astype(jnp.int32)
             .reshape(NB, 128, L).transpose(0, 2, 1).reshape(-1))
  arc_idx = (deparc_path_edge.astype(jnp.int32)
             .reshape(NB, 128, L).transpose(0, 2, 1).reshape(-1))
  pair_rel = (deprel_table[:, None, :] + deprel_table[None, :, :]).reshape(-1)
  pair_rel = jnp.pad(pair_rel, (0, (PAIR_ROWS - 2500) * DIM))
  s2 = (deparc_table[:, None, :] + deparc_table[None, :, :]).reshape(16, DIM)
  quad_arc = (s2[:, None, :] + s2[None, :, :]).reshape(-1)
  out = _encode(rel_idx, arc_idx, pair_rel, quad_arc)
  return out.reshape(4, NB, 8, 128).transpose(1, 3, 0, 2).reshape(N, 32)
